# baseline (speedup 1.0000x reference)
import functools

import jax
import jax.numpy as jnp
from jax.experimental import pallas as pl
from jax.experimental.layout import Format, Layout

_K = 8


def _router_block_kernel(x_ref, w1_ref, b1_ref, w2_ref, b2_ref,
                         logits_ref, weights_ref, indices_ref, masks_ref):
    h = jnp.dot(x_ref[...], w1_ref[...], preferred_element_type=jnp.float32)
    h = h + b1_ref[...]
    logits = jnp.dot(h, w2_ref[...], preferred_element_type=jnp.float32)
    logits = logits + b2_ref[...]
    logits_ref[...] = logits

    lt = logits.T
    e, nb = lt.shape
    iota_s = jax.lax.broadcasted_iota(jnp.int32, (e, nb), 0)
    work = lt
    vals, idxs = [], []
    for _ in range(_K):
        m = jnp.max(work, axis=0, keepdims=True)
        hit = work == m
        idx = jnp.min(jnp.where(hit, iota_s, e), axis=0, keepdims=True)
        sel = iota_s == idx
        work = jnp.where(sel, -jnp.inf, work)
        vals.append(m)
        idxs.append(idx)
    vals_t = jnp.concatenate(vals, axis=0)
    idxs_t = jnp.concatenate(idxs, axis=0)

    w = jnp.exp(vals_t - vals_t[0:1])
    wn = w / jnp.sum(w, axis=0, keepdims=True)
    weights_ref[...] = wn.T
    indices_ref[...] = idxs_t.T

    iota_e3 = jax.lax.broadcasted_iota(jnp.int32, (e, _K, nb), 0)
    masks_ref[...] = (iota_e3 == idxs_t[None, :, :]).astype(jnp.int32)


def kernel_impl(x, W1, b1, W2, b2):
    n, d = x.shape
    h_dim = W1.shape[1]
    e = W2.shape[1]
    nb = 1024 if n % 1024 == 0 else n
    grid = (n // nb,)
    out_shapes = (
        jax.ShapeDtypeStruct((n, e), jnp.float32),
        jax.ShapeDtypeStruct((n, _K), jnp.float32),
        jax.ShapeDtypeStruct((n, _K), jnp.int32),
        jax.ShapeDtypeStruct((e, _K, n), jnp.int32),
    )
    return pl.pallas_call(
        _router_block_kernel,
        grid=grid,
        in_specs=[
            pl.BlockSpec((nb, d), lambda i: (i, 0)),
            pl.BlockSpec((d, h_dim), lambda i: (0, 0)),
            pl.BlockSpec((1, h_dim), lambda i: (0, 0)),
            pl.BlockSpec((h_dim, e), lambda i: (0, 0)),
            pl.BlockSpec((1, e), lambda i: (0, 0)),
        ],
        out_specs=(
            pl.BlockSpec((nb, e), lambda i: (i, 0)),
            pl.BlockSpec((nb, _K), lambda i: (i, 0)),
            pl.BlockSpec((nb, _K), lambda i: (i, 0)),
            pl.BlockSpec((e, _K, nb), lambda i: (0, 0, i)),
        ),
        out_shape=out_shapes,
    )(x, W1, b1.reshape(1, -1), W2, b2.reshape(1, -1))


_OUT_LAYOUTS = (
    Layout(major_to_minor=(0, 1), tiling=((8, 128),)),
    Layout(major_to_minor=(0, 1), tiling=((8, 128),)),
    Layout(major_to_minor=(0, 1), tiling=((8, 128),)),
    Layout(major_to_minor=(0, 1, 2), tiling=((8, 128),)),
)


@functools.lru_cache(maxsize=None)
def _jitted_for(dev):
    if dev is not None and dev.platform == "tpu":
        sh = jax.sharding.SingleDeviceSharding(dev)
        fmts = tuple(Format(l, sh) for l in _OUT_LAYOUTS)
        return jax.jit(kernel_impl, out_shardings=fmts)
    return jax.jit(kernel_impl)


def kernel(x, W1, b1, W2, b2):
    try:
        dev = next(iter(x.devices()))
    except Exception:
        dev = None
    return _jitted_for(dev)(x, W1, b1, W2, b2)


# speedup vs baseline: 1.5247x; 1.5247x over previous
import jax
import jax.numpy as jnp
from jax.experimental import pallas as pl

_K = 8


def _router_block_kernel(x_ref, w1_ref, b1_ref, w2t_ref, b2_ref,
                         logits_t_ref, weights_t_ref, indices_t_ref,
                         masks_ref):
    h = jnp.dot(x_ref[...], w1_ref[...], preferred_element_type=jnp.float32)
    h = h + b1_ref[...][None, :]
    logits = jax.lax.dot_general(
        h, w2t_ref[...], (((1,), (1,)), ((), ())),
        preferred_element_type=jnp.float32)
    logits = logits + b2_ref[...][None, :]

    lt = logits.T
    e, nb = lt.shape
    logits_t_ref[...] = lt

    iota_s = jax.lax.broadcasted_iota(jnp.int32, (e, nb), 0)
    work = lt
    vals, idxs = [], []
    for _ in range(_K):
        m = jnp.max(work, axis=0, keepdims=True)
        hit = work == m
        idx = jnp.min(jnp.where(hit, iota_s, e), axis=0, keepdims=True)
        sel = iota_s == idx
        work = jnp.where(sel, -jnp.inf, work)
        vals.append(m)
        idxs.append(idx)
    vals_t = jnp.concatenate(vals, axis=0)
    idxs_t = jnp.concatenate(idxs, axis=0)

    w = jnp.exp(vals_t - vals_t[0:1])
    weights_t_ref[...] = w / jnp.sum(w, axis=0, keepdims=True)
    indices_t_ref[...] = idxs_t

    iota_e3 = jax.lax.broadcasted_iota(jnp.int32, (e, _K, nb), 0)
    masks_ref[...] = (iota_e3 == idxs_t[None, :, :]).astype(jnp.int32)


@jax.jit
def kernel(x, W1, b1, W2, b2):
    n, d = x.shape
    h_dim = W1.shape[1]
    e = W2.shape[1]
    nb = 1024 if n % 1024 == 0 else n
    grid = (n // nb,)
    out_shapes = (
        jax.ShapeDtypeStruct((e, n), jnp.float32),
        jax.ShapeDtypeStruct((_K, n), jnp.float32),
        jax.ShapeDtypeStruct((_K, n), jnp.int32),
        jax.ShapeDtypeStruct((e, _K, n), jnp.int32),
    )
    logits_t, weights_t, indices_t, masks = pl.pallas_call(
        _router_block_kernel,
        grid=grid,
        in_specs=[
            pl.BlockSpec((nb, d), lambda i: (i, 0)),
            pl.BlockSpec((d, h_dim), lambda i: (0, 0)),
            pl.BlockSpec((h_dim,), lambda i: (0,)),
            pl.BlockSpec((e, h_dim), lambda i: (0, 0)),
            pl.BlockSpec((e,), lambda i: (0,)),
        ],
        out_specs=(
            pl.BlockSpec((e, nb), lambda i: (0, i)),
            pl.BlockSpec((_K, nb), lambda i: (0, i)),
            pl.BlockSpec((_K, nb), lambda i: (0, i)),
            pl.BlockSpec((e, _K, nb), lambda i: (0, 0, i)),
        ),
        out_shape=out_shapes,
    )(x, W1, b1, W2.T, b2)
    return (logits_t.T, weights_t.T, indices_t.T, masks)


# speedup vs baseline: 1.7493x; 1.1473x over previous
import jax
import jax.numpy as jnp
from jax.experimental import pallas as pl

_K = 8


def _router_block_kernel(x_ref, w1_ref, b1_ref, w2t_ref, b2_ref,
                         logits_t_ref, weights_t_ref, indices_t_ref,
                         masks_ref):
    h = jnp.dot(x_ref[...], w1_ref[...], preferred_element_type=jnp.float32)
    h = h + b1_ref[...][None, :]
    logits = jax.lax.dot_general(
        h, w2t_ref[...], (((1,), (1,)), ((), ())),
        preferred_element_type=jnp.float32)
    logits = logits + b2_ref[...][None, :]

    lt = logits.T
    e, nb = lt.shape
    logits_t_ref[...] = lt

    iota_s = jax.lax.broadcasted_iota(jnp.int32, (e, nb), 0)
    work = lt
    vals, idxs = [], []
    for _ in range(_K):
        m = jnp.max(work, axis=0, keepdims=True)
        hit = work == m
        idx = jnp.min(jnp.where(hit, iota_s, e), axis=0, keepdims=True)
        sel = iota_s == idx
        work = jnp.where(sel, -jnp.inf, work)
        vals.append(m)
        idxs.append(idx)
    vals_t = jnp.concatenate(vals, axis=0)
    idxs_t = jnp.concatenate(idxs, axis=0)

    w = jnp.exp(vals_t - vals_t[0:1])
    weights_t_ref[...] = w / jnp.sum(w, axis=0, keepdims=True)
    indices_t_ref[...] = idxs_t

    iota_e3 = jax.lax.broadcasted_iota(jnp.int32, (e, _K, nb), 0)
    masks_ref[...] = (iota_e3 == idxs_t[None, :, :]).astype(jnp.int32)


@jax.jit
def kernel(x, W1, b1, W2, b2):
    n, d = x.shape
    h_dim = W1.shape[1]
    e = W2.shape[1]
    nb = 2048 if n % 2048 == 0 else n
    grid = (n // nb,)
    out_shapes = (
        jax.ShapeDtypeStruct((e, n), jnp.float32),
        jax.ShapeDtypeStruct((_K, n), jnp.float32),
        jax.ShapeDtypeStruct((_K, n), jnp.int32),
        jax.ShapeDtypeStruct((e, _K, n), jnp.int32),
    )
    logits_t, weights_t, indices_t, masks = pl.pallas_call(
        _router_block_kernel,
        grid=grid,
        in_specs=[
            pl.BlockSpec((nb, d), lambda i: (i, 0)),
            pl.BlockSpec((d, h_dim), lambda i: (0, 0)),
            pl.BlockSpec((h_dim,), lambda i: (0,)),
            pl.BlockSpec((e, h_dim), lambda i: (0, 0)),
            pl.BlockSpec((e,), lambda i: (0,)),
        ],
        out_specs=(
            pl.BlockSpec((e, nb), lambda i: (0, i)),
            pl.BlockSpec((_K, nb), lambda i: (0, i)),
            pl.BlockSpec((_K, nb), lambda i: (0, i)),
            pl.BlockSpec((e, _K, nb), lambda i: (0, 0, i)),
        ),
        out_shape=out_shapes,
    )(x, W1, b1, W2.T, b2)
    return (logits_t.T, weights_t.T, indices_t.T, masks)


# speedup vs baseline: 1.8369x; 1.0501x over previous
import jax
import jax.numpy as jnp
from jax.experimental import pallas as pl

_K = 8


def _router_block_kernel(x_ref, w1_ref, b1_ref, w2t_ref, b2_ref,
                         logits_t_ref, weights_t_ref, indices_t_ref,
                         masks_ref):
    h = jnp.dot(x_ref[...], w1_ref[...], preferred_element_type=jnp.float32)
    h = h + b1_ref[...][None, :]
    logits = jax.lax.dot_general(
        h, w2t_ref[...], (((1,), (1,)), ((), ())),
        preferred_element_type=jnp.float32)
    logits = logits + b2_ref[...][None, :]

    lt = logits.T
    e, nb = lt.shape
    logits_t_ref[...] = lt

    iota_s = jax.lax.broadcasted_iota(jnp.int32, (e, nb), 0)
    work = lt
    vals, idxs = [], []
    for _ in range(_K):
        m = jnp.max(work, axis=0, keepdims=True)
        hit = work == m
        idx = jnp.min(jnp.where(hit, iota_s, e), axis=0, keepdims=True)
        sel = iota_s == idx
        work = jnp.where(sel, -jnp.inf, work)
        vals.append(m)
        idxs.append(idx)
    vals_t = jnp.concatenate(vals, axis=0)
    idxs_t = jnp.concatenate(idxs, axis=0)

    w = jnp.exp(vals_t - vals_t[0:1])
    weights_t_ref[...] = w / jnp.sum(w, axis=0, keepdims=True)
    indices_t_ref[...] = idxs_t

    iota_e3 = jax.lax.broadcasted_iota(jnp.int32, (e, _K, nb), 0)
    masks_ref[...] = (iota_e3 == idxs_t[None, :, :]).astype(jnp.int32)


@jax.jit
def kernel(x, W1, b1, W2, b2):
    n, d = x.shape
    h_dim = W1.shape[1]
    e = W2.shape[1]
    nb = 4096 if n % 4096 == 0 else n
    grid = (n // nb,)
    out_shapes = (
        jax.ShapeDtypeStruct((e, n), jnp.float32),
        jax.ShapeDtypeStruct((_K, n), jnp.float32),
        jax.ShapeDtypeStruct((_K, n), jnp.int32),
        jax.ShapeDtypeStruct((e, _K, n), jnp.int32),
    )
    logits_t, weights_t, indices_t, masks = pl.pallas_call(
        _router_block_kernel,
        grid=grid,
        in_specs=[
            pl.BlockSpec((nb, d), lambda i: (i, 0)),
            pl.BlockSpec((d, h_dim), lambda i: (0, 0)),
            pl.BlockSpec((h_dim,), lambda i: (0,)),
            pl.BlockSpec((e, h_dim), lambda i: (0, 0)),
            pl.BlockSpec((e,), lambda i: (0,)),
        ],
        out_specs=(
            pl.BlockSpec((e, nb), lambda i: (0, i)),
            pl.BlockSpec((_K, nb), lambda i: (0, i)),
            pl.BlockSpec((_K, nb), lambda i: (0, i)),
            pl.BlockSpec((e, _K, nb), lambda i: (0, 0, i)),
        ),
        out_shape=out_shapes,
    )(x, W1, b1, W2.T, b2)
    return (logits_t.T, weights_t.T, indices_t.T, masks)
